# Initial kernel scaffold; baseline (speedup 1.0000x reference)
import jax
import jax.numpy as jnp
from jax.experimental import pallas as pl


def kernel(x_seq, params):
    raise NotImplementedError("write your pallas kernel here")


# speedup vs baseline: 6.3444x; 6.3444x over previous
import math

import jax
import jax.numpy as jnp
from jax.experimental import pallas as pl
from jax.experimental.pallas import tpu as pltpu

D = 64
V = 8192
B = 16
T = 32
SLOTS = 32
MAX_REC = 8
PONDER_COST = 0.01
VCHUNK = 512
NCHUNK = V // VCHUNK

_ACT, _VQ, _ENT, _POND, _PH = 0, 1, 2, 3, 4


def _body(x_seq_ref, scal_ref, enc_ref, vq_ref, esq_ref,
          qW_ref, kW_ref, vW_ref, qb_ref, kb_ref, vb_ref,
          arbW_ref, arbb_ref, gateW_ref, lng_ref, lnb_ref,
          decW_ref, decb_ref,
          logits_ref, stats_ref, idx_ref,
          memc_ref, flat_ref, xb_ref, gw_ref, acc_ref, fin_ref, pang_ref):
    alpha = scal_ref[0, 0]
    hbias = scal_ref[0, 1]
    gate_b = scal_ref[0, 2]

    memc_ref[...] = jnp.zeros_like(memc_ref)
    idx_ref[...] = jnp.zeros_like(idx_ref)
    gw_ref[...] = jnp.zeros_like(gw_ref)

    lane128 = jax.lax.broadcasted_iota(jnp.int32, (B, 2 * D), 1)
    maskr = (lane128 < D).astype(jnp.float32)
    maski = 1.0 - maskr
    lng = lng_ref[...]
    lnb = lnb_ref[...]
    qW = qW_ref[...]
    kW = kW_ref[...]
    vW = vW_ref[...]
    arbW = arbW_ref[...]
    arbb = arbb_ref[...]
    gateW = gateW_ref[...]
    acc_init = (jax.lax.broadcasted_iota(jnp.int32, (B, 8), 1)
                == _ACT).astype(jnp.float32)

    def ln_concat(gwc):
        m_r = jnp.sum(gwc * maskr, axis=-1, keepdims=True) * (1.0 / D)
        m_i = jnp.sum(gwc * maski, axis=-1, keepdims=True) * (1.0 / D)
        m_sel = jnp.where(lane128 < D, m_r, m_i)
        xc = gwc - m_sel
        x2 = xc * xc
        v_r = jnp.sum(x2 * maskr, axis=-1, keepdims=True) * (1.0 / D)
        v_i = jnp.sum(x2 * maski, axis=-1, keepdims=True) * (1.0 / D)
        v_sel = jnp.where(lane128 < D, v_r, v_i)
        return xc / jnp.sqrt(v_sel + 1e-5) * lng + lnb

    def vq_search(z):
        zsq = jnp.sum(z * z, axis=-1, keepdims=True)
        big = jnp.float32(1e30)

        def chunk(c, carry):
            best, bidx, zq = carry
            ec = vq_ref[pl.ds(c * VCHUNK, VCHUNK), :]
            s = jax.lax.dot_general(z, ec, (((1,), (1,)), ((), ())),
                                    preferred_element_type=jnp.float32)
            d2 = (zsq - 2.0 * s) + esq_ref[:, pl.ds(c * VCHUNK, VCHUNK)]
            lmin = jnp.min(d2, axis=-1, keepdims=True)
            lio = (jax.lax.broadcasted_iota(jnp.int32, (B, VCHUNK), 1)
                   + c * VCHUNK)
            lidx = jnp.min(jnp.where(d2 == lmin, lio, jnp.int32(2 ** 30)),
                           axis=-1, keepdims=True)
            oh = (lio == lidx).astype(jnp.float32)
            zql = jax.lax.dot_general(oh, ec, (((1,), (0,)), ((), ())),
                                      precision=jax.lax.Precision.HIGHEST,
                                      preferred_element_type=jnp.float32)
            take = lmin < best
            best = jnp.where(take, lmin, best)
            bidx = jnp.where(take, lidx, bidx)
            zq = jnp.where(take, zql, zq)
            return best, bidx, zq

        z0 = z[:, 0:1]
        init = (jnp.where(jnp.abs(z0) > big, z0, big),
                (z0 > big).astype(jnp.int32),
                jnp.where(jnp.abs(z) > big, z, 0.0))
        _, bidx, zq = jax.lax.fori_loop(0, NCHUNK, chunk, init)
        return bidx, zq

    def entropy(bidx):
        def entchunk(c, cb):
            lio = (jax.lax.broadcasted_iota(jnp.int32, (B, VCHUNK), 1)
                   + c * VCHUNK)
            ohc = (lio == bidx).astype(jnp.float32)
            cnt = jnp.sum(ohc, axis=0, keepdims=True)
            return cb + jnp.sum(ohc * cnt, axis=-1, keepdims=True)

        c_b = jax.lax.fori_loop(0, NCHUNK, entchunk,
                                (bidx < 0).astype(jnp.float32))
        logs = jnp.log(c_b * (1.0 / B) + 1e-10)
        return -jnp.sum(logs, axis=0, keepdims=True) * (1.0 / B)

    def rec_step(st):
        it, _ = st
        gwc = gw_ref[...]
        active = acc_ref[:, _ACT:_ACT + 1]
        pang = pang_ref[...]

        z = ln_concat(gwc)
        bidx, zq = vq_search(z)
        zqst = z + (zq - z)
        dqz = zq - z
        a = jnp.sum(dqz * dqz, axis=-1, keepdims=True) * (1.0 / (2 * D))
        vq_loss = a + 0.25 * a
        ent = entropy(bidx)

        q = jnp.dot(z, qW, preferred_element_type=jnp.float32) + qb_ref[...]
        k = jnp.dot(z, kW, preferred_element_type=jnp.float32) + kb_ref[...]
        v = jnp.dot(z, vW, preferred_element_type=jnp.float32) + vb_ref[...]
        gate = jax.nn.sigmoid(jnp.sum(q * k, axis=-1, keepdims=True))
        g = v * gate

        mem = memc_ref[...]
        sim = jnp.sum(mem * z[:, None, :], axis=-1)
        attn = jax.nn.softmax(sim, axis=-1)
        m = jnp.sum(mem * attn[:, :, None], axis=1)

        ga = jax.nn.softmax(
            jnp.dot(z, arbW, preferred_element_type=jnp.float32) + arbb,
            axis=-1)
        up = ga[:, 0:1] * g + ga[:, 1:2] * m + ga[:, 2:3] * zqst
        cand = 0.6 * z + 0.4 * up

        ang = jnp.arctan2(cand[:, D:], cand[:, :D])
        diff = jnp.abs(ang - pang)
        diff = jnp.minimum(diff, 2.0 * math.pi - diff)
        acc_ref[:, _PH:_PH + 1] += active * (
            jnp.sum(diff, axis=-1, keepdims=True) * (1.0 / D))
        pang_ref[...] = ang

        stop = (hbias - vq_loss > 0.0).astype(jnp.float32)
        acc_ref[:, _POND:_POND + 1] += active * PONDER_COST
        maskf = active > 0.5
        acc_ref[:, _VQ:_VQ + 1] = jnp.where(
            maskf, vq_loss, acc_ref[:, _VQ:_VQ + 1])
        acc_ref[:, _ENT:_ENT + 1] = jnp.where(
            maskf, jnp.broadcast_to(ent, (B, 1)), acc_ref[:, _ENT:_ENT + 1])
        fin_ref[...] = jnp.where(maskf, bidx, fin_ref[...])
        gw_ref[...] = jnp.where(maskf, cand, gwc)
        new_active = active * (1.0 - stop)
        acc_ref[:, _ACT:_ACT + 1] = new_active
        return it + 1, jnp.max(new_active) > 0.5

    def rec_cond(st):
        it, go = st
        return jnp.logical_and(it < MAX_REC, go)

    lane32 = jax.lax.broadcasted_iota(jnp.int32, (B, T), 1)

    def tstep(t, carry):
        s0, s1, s2, s3 = carry
        for b in range(B):
            xb_ref[pl.ds(b, 1), :] = enc_ref[pl.ds(x_seq_ref[b, t], 1), :]
        xc = xb_ref[...]
        gwc = alpha * gw_ref[...] + (1.0 - alpha) * xc
        gw_ref[...] = gwc

        pang_ref[...] = jnp.arctan2(gwc[:, D:], gwc[:, :D])
        acc_ref[...] = acc_init
        fin_ref[...] = (gwc[:, 0:1] > jnp.float32(1e30)).astype(jnp.int32)
        jax.lax.while_loop(rec_cond, rec_step, (jnp.int32(0), True))

        gwc = gw_ref[...]
        wg = jax.nn.sigmoid(jnp.sum(gwc * gateW, axis=-1, keepdims=True)
                            + gate_b)
        mem = memc_ref[...]
        last = mem[:, SLOTS - 1, :]
        head0 = wg * gwc + (1.0 - wg) * last
        memc_ref[...] = jnp.concatenate(
            [head0[:, None, :], mem[:, :SLOTS - 1, :]], axis=1)

        flat_ref[pl.ds(pl.multiple_of(t * B, B), B), :] = gwc
        idx_ref[...] = jnp.where(
            lane32 == t, jnp.broadcast_to(fin_ref[...], (B, T)), idx_ref[...])

        s0 = s0 + jnp.sum(acc_ref[:, _VQ:_VQ + 1], axis=0, keepdims=True)
        s1 = s1 + jnp.sum(acc_ref[:, _ENT:_ENT + 1], axis=0, keepdims=True)
        s2 = s2 + jnp.sum(acc_ref[:, _POND:_POND + 1], axis=0, keepdims=True)
        s3 = s3 + jnp.sum(acc_ref[:, _PH:_PH + 1], axis=0, keepdims=True)
        return s0, s1, s2, s3

    z11 = jnp.zeros((1, 1), jnp.float32)
    s0, s1, s2, s3 = jax.lax.fori_loop(0, T, tstep, (z11, z11, z11, z11))

    stats_ref[...] = (jnp.concatenate([s0, s1, s2, s3], axis=1)
                      * (1.0 / (B * T)))
    fl = flat_ref[...]
    logits_ref[...] = jax.lax.dot_general(
        fl, decW_ref[...], (((1,), (1,)), ((), ())),
        preferred_element_type=jnp.float32) + decb_ref[...]


def kernel(x_seq, params):
    p = params
    f32 = jnp.float32
    qWc = jnp.block([[p['qW_r'].T, p['qW_i'].T], [-p['qW_i'].T, p['qW_r'].T]])
    kWc = jnp.block([[p['kW_r'].T, p['kW_i'].T], [-p['kW_i'].T, p['kW_r'].T]])
    vWc = jnp.block([[p['vW_r'].T, p['vW_i'].T], [-p['vW_i'].T, p['vW_r'].T]])
    qbc = jnp.concatenate([p['qb_r'] - p['qb_i'], p['qb_r'] + p['qb_i']])[None]
    kbc = jnp.concatenate([p['kb_r'] - p['kb_i'], p['kb_r'] + p['kb_i']])[None]
    vbc = jnp.concatenate([p['vb_r'] - p['vb_i'], p['vb_r'] + p['vb_i']])[None]
    esq = (p['vq_emb'] ** 2).sum(-1)[None, :]
    scal = jnp.stack([jax.nn.sigmoid(p['input_gate']),
                      jax.nn.softplus(p['halt_bias']),
                      p['gate_b'][0], jnp.float32(0.0)])[None].astype(f32)
    lngc = jnp.concatenate([p['ln_r_g'], p['ln_i_g']])[None]
    lnbc = jnp.concatenate([p['ln_r_b'], p['ln_i_b']])[None]

    logits_tm, stats, idx = pl.pallas_call(
        _body,
        out_shape=(
            jax.ShapeDtypeStruct((B * T, V), f32),
            jax.ShapeDtypeStruct((1, 4), f32),
            jax.ShapeDtypeStruct((B, T), jnp.int32),
        ),
        in_specs=[
            pl.BlockSpec(memory_space=pltpu.SMEM),
            pl.BlockSpec(memory_space=pltpu.SMEM),
        ] + [pl.BlockSpec()] * 16,
        scratch_shapes=[
            pltpu.VMEM((B, SLOTS, 2 * D), f32),
            pltpu.VMEM((B * T, 2 * D), f32),
            pltpu.VMEM((B, 2 * D), f32),
            pltpu.VMEM((B, 2 * D), f32),
            pltpu.VMEM((B, 8), f32),
            pltpu.VMEM((B, 1), jnp.int32),
            pltpu.VMEM((B, D), f32),
        ],
        compiler_params=pltpu.CompilerParams(
            vmem_limit_bytes=60 * 1024 * 1024),
    )(x_seq, scal, p['enc'], p['vq_emb'], esq,
      qWc, kWc, vWc, qbc, kbc, vbc,
      p['arb_W'].T, p['arb_b'][None], p['gate_W'],
      lngc, lnbc, p['dec_W'], p['dec_b'][None])

    logits = logits_tm.reshape(T, B, V).transpose(1, 0, 2)
    return logits, stats.reshape(4), idx
